# Initial kernel scaffold; baseline (speedup 1.0000x reference)
import functools

import jax
import jax.numpy as jnp
from jax import lax
from jax.experimental import pallas as pl
from jax.experimental.pallas import tpu as pltpu
from jax.experimental.pallas import tpu_sc as plsc

N = 10000
D = 128
E = 320000

NC = 2
NS = 16
NW = NC * NS
CHUNK = 128
NPAD = N + 16
ROWS_PT = NPAD // NS
E_PAD = ((E + NW * CHUNK - 1) // (NW * CHUNK)) * (NW * CHUNK)
CPT = E_PAD // (NW * CHUNK)
DEGC = 16

_MESH = plsc.VectorSubcoreMesh(core_axis_name="c", subcore_axis_name="s")


@functools.partial(
    pl.kernel,
    out_type=jax.ShapeDtypeStruct((NC, NPAD, DEGC), jnp.float32),
    mesh=_MESH,
    scratch_types=[
        pltpu.VMEM((CHUNK,), jnp.int32),
        pltpu.VMEM((CHUNK, DEGC), jnp.float32),
        pltpu.VMEM_SHARED((NPAD, DEGC), jnp.float32),
    ],
)
def _sc_degree(src_hbm, ones_hbm, zeros_hbm, out_hbm, sidx, ones_v, acc):
    cid = lax.axis_index("c")
    sid = lax.axis_index("s")
    wid = sid * NC + cid
    r0 = sid * ROWS_PT
    pltpu.sync_copy(zeros_hbm.at[pl.ds(r0, ROWS_PT)], acc.at[pl.ds(r0, ROWS_PT)])
    pltpu.sync_copy(ones_hbm, ones_v)
    plsc.subcore_barrier()
    base = wid * CPT * CHUNK

    @pl.loop(0, CPT)
    def _(k):
        pltpu.sync_copy(src_hbm.at[pl.ds(base + k * CHUNK, CHUNK)], sidx)
        pltpu.sync_copy(ones_v, acc.at[sidx], add=True)

    plsc.subcore_barrier()
    pltpu.sync_copy(acc.at[pl.ds(r0, ROWS_PT)],
                    out_hbm.at[cid].at[pl.ds(r0, ROWS_PT)])


@functools.partial(
    pl.kernel,
    out_type=jax.ShapeDtypeStruct((NC, NPAD, D), jnp.float32),
    mesh=_MESH,
    scratch_types=[
        pltpu.VMEM((CHUNK,), jnp.int32),
        pltpu.VMEM((CHUNK,), jnp.int32),
        pltpu.VMEM((CHUNK, D), jnp.float32),
        pltpu.VMEM_SHARED((NPAD, D), jnp.float32),
        pltpu.SemaphoreType.DMA,
    ],
)
def _sc_messages(h_hbm, src_hbm, dst_hbm, zeros_hbm, out_hbm,
                 sidx, didx, rows, acc, sem):
    cid = lax.axis_index("c")
    sid = lax.axis_index("s")
    wid = sid * NC + cid
    r0 = sid * ROWS_PT
    pltpu.sync_copy(zeros_hbm.at[pl.ds(r0, ROWS_PT)], acc.at[pl.ds(r0, ROWS_PT)])
    plsc.subcore_barrier()
    base = wid * CPT * CHUNK

    @pl.loop(0, CPT)
    def _(k):
        off = base + k * CHUNK
        pltpu.sync_copy(src_hbm.at[pl.ds(off, CHUNK)], sidx)
        pltpu.sync_copy(dst_hbm.at[pl.ds(off, CHUNK)], didx)
        pltpu.async_copy(h_hbm.at[sidx], rows, sem).wait()
        pltpu.sync_copy(rows, acc.at[didx], add=True)

    plsc.subcore_barrier()
    pltpu.sync_copy(acc.at[pl.ds(r0, ROWS_PT)],
                    out_hbm.at[cid].at[pl.ds(r0, ROWS_PT)])


def _dinv_cols(degp):
    deg = degp[0, :N, :1] + degp[1, :N, :1] + 1.0
    return lax.rsqrt(deg)


def _leaky(v):
    return jnp.where(v >= 0, v, 0.1 * v)


def _scaled_matmul_pad(xs, w, out_ref):
    h = lax.dot_general(xs, w, (((1,), (1,)), ((), ())),
                        preferred_element_type=jnp.float32)
    out_ref[:N, :] = h
    out_ref[N:, :] = jnp.zeros((NPAD - N, D), jnp.float32)


def _tc1_body(x_ref, w1_ref, degp_ref, h1_ref):
    dinv = _dinv_cols(degp_ref[...])
    _scaled_matmul_pad(x_ref[...] * dinv, w1_ref[...], h1_ref)


def _block_tail(sp, hpad, x_in, degp, b, g, be):
    dinv = _dinv_cols(degp)
    s = sp[0, :N, :] + sp[1, :N, :]
    out = dinv * (s + hpad[:N, :]) + b[None, :]
    mean = jnp.mean(out, axis=0)
    var = jnp.mean((out - mean[None, :]) ** 2, axis=0)
    z = (out - mean[None, :]) * lax.rsqrt(var[None, :] + 1e-5) * g[None, :] + be[None, :]
    return _leaky(_leaky(z) + x_in), dinv


def _tc2_body(s1_ref, h1_ref, x_ref, degp_ref, b1_ref, g1_ref, be1_ref,
              w2_ref, x2_ref, h2_ref):
    x2, dinv = _block_tail(s1_ref[...], h1_ref[...], x_ref[...], degp_ref[...],
                           b1_ref[...], g1_ref[...], be1_ref[...])
    x2_ref[...] = x2
    _scaled_matmul_pad(x2 * dinv, w2_ref[...], h2_ref)


def _tc3_body(s2_ref, h2_ref, x2_ref, degp_ref, b2_ref, g2_ref, be2_ref,
              out_ref):
    h, _ = _block_tail(s2_ref[...], h2_ref[...], x2_ref[...], degp_ref[...],
                       b2_ref[...], g2_ref[...], be2_ref[...])
    nrm = jnp.maximum(jnp.sqrt(jnp.sum(h * h, axis=1, keepdims=True)), 1e-12)
    out_ref[...] = h / nrm


_tc1 = pl.pallas_call(
    _tc1_body,
    out_shape=jax.ShapeDtypeStruct((NPAD, D), jnp.float32),
)

_tc2 = pl.pallas_call(
    _tc2_body,
    out_shape=(jax.ShapeDtypeStruct((N, D), jnp.float32),
               jax.ShapeDtypeStruct((NPAD, D), jnp.float32)),
)

_tc3 = pl.pallas_call(
    _tc3_body,
    out_shape=jax.ShapeDtypeStruct((N, D), jnp.float32),
)


def kernel(x, edge_index, W1, b1, g1, be1, W2, b2, g2, be2):
    src = edge_index[0]
    dst = edge_index[1]
    pad = jnp.full((E_PAD - E,), N, jnp.int32)
    srcp = jnp.concatenate([src, pad])
    dstp = jnp.concatenate([dst, pad])
    zeros_deg = jnp.zeros((NPAD, DEGC), jnp.float32)
    ones_deg = jnp.ones((CHUNK, DEGC), jnp.float32)
    zeros_d = jnp.zeros((NPAD, D), jnp.float32)

    degp = _sc_degree(srcp, ones_deg, zeros_deg)
    h1 = _tc1(x, W1, degp)
    s1 = _sc_messages(h1, srcp, dstp, zeros_d)
    x2, h2 = _tc2(s1, h1, x, degp, b1, g1, be1, W2)
    s2 = _sc_messages(h2, srcp, dstp, zeros_d)
    return _tc3(s2, h2, x2, degp, b2, g2, be2)


# speedup vs baseline: 9.3814x; 9.3814x over previous
import functools

import jax
import jax.numpy as jnp
from jax import lax
from jax.experimental import pallas as pl
from jax.experimental.pallas import tpu as pltpu
from jax.experimental.pallas import tpu_sc as plsc

N = 10000
D = 128
E = 320000

NC = 2
NS = 16
NW = NC * NS
CHUNK = 128
NPAD = 10112
ROWS_PT = NPAD // NS
E_PAD = ((E + NW * CHUNK - 1) // (NW * CHUNK)) * (NW * CHUNK)
CPT = E_PAD // (NW * CHUNK)
DEGC = 128

_MESH = plsc.VectorSubcoreMesh(core_axis_name="c", subcore_axis_name="s")


@functools.partial(
    pl.kernel,
    out_type=jax.ShapeDtypeStruct((NC, NPAD, DEGC), jnp.float32),
    mesh=_MESH,
    scratch_types=[
        pltpu.VMEM((CHUNK,), jnp.int32),
        pltpu.VMEM((CHUNK, DEGC), jnp.float32),
        pltpu.VMEM_SHARED((NPAD, DEGC), jnp.float32),
    ],
)
def _sc_degree(src_hbm, ones_hbm, zeros_hbm, out_hbm, sidx, ones_v, acc):
    cid = lax.axis_index("c")
    sid = lax.axis_index("s")
    wid = sid * NC + cid
    r0 = sid * ROWS_PT
    pltpu.sync_copy(zeros_hbm.at[pl.ds(r0, ROWS_PT)], acc.at[pl.ds(r0, ROWS_PT)])
    pltpu.sync_copy(ones_hbm, ones_v)
    plsc.subcore_barrier()
    base = wid * CPT * CHUNK

    @pl.loop(0, CPT)
    def _(k):
        pltpu.sync_copy(src_hbm.at[pl.ds(base + k * CHUNK, CHUNK)], sidx)
        pltpu.sync_copy(ones_v, acc.at[sidx], add=True)


    plsc.subcore_barrier()
    pltpu.sync_copy(acc.at[pl.ds(r0, ROWS_PT)],
                    out_hbm.at[cid].at[pl.ds(r0, ROWS_PT)])


@functools.partial(
    pl.kernel,
    out_type=jax.ShapeDtypeStruct((NC, NPAD, D), jnp.float32),
    mesh=_MESH,
    scratch_types=[
        pltpu.VMEM((CHUNK,), jnp.int32),
        pltpu.VMEM((CHUNK,), jnp.int32),
        pltpu.VMEM((CHUNK, D), jnp.float32),
        pltpu.VMEM_SHARED((NPAD, D), jnp.float32),
        pltpu.SemaphoreType.DMA,
    ],
)
def _sc_messages(h_hbm, src_hbm, dst_hbm, zeros_hbm, out_hbm,
                 sidx, didx, rows, acc, sem):
    cid = lax.axis_index("c")
    sid = lax.axis_index("s")
    wid = sid * NC + cid
    r0 = sid * ROWS_PT
    pltpu.sync_copy(zeros_hbm.at[pl.ds(r0, ROWS_PT)], acc.at[pl.ds(r0, ROWS_PT)])
    plsc.subcore_barrier()
    base = wid * CPT * CHUNK

    @pl.loop(0, CPT)
    def _(k):
        off = base + k * CHUNK
        pltpu.sync_copy(src_hbm.at[pl.ds(off, CHUNK)], sidx)
        pltpu.sync_copy(dst_hbm.at[pl.ds(off, CHUNK)], didx)
        pltpu.async_copy(h_hbm.at[sidx], rows, sem).wait()
        pltpu.sync_copy(rows, acc.at[didx], add=True)

    plsc.subcore_barrier()
    pltpu.sync_copy(acc.at[pl.ds(r0, ROWS_PT)],
                    out_hbm.at[cid].at[pl.ds(r0, ROWS_PT)])


def _dinv_cols(degp):
    deg = degp[0, :N, :1] + degp[1, :N, :1] + 1.0
    return lax.rsqrt(deg)


def _leaky(v):
    return jnp.where(v >= 0, v, 0.1 * v)


def _scaled_matmul_pad(xs, w, out_ref):
    h = lax.dot_general(xs, w, (((1,), (1,)), ((), ())),
                        preferred_element_type=jnp.float32)
    out_ref[:N, :] = h
    out_ref[N:, :] = jnp.zeros((NPAD - N, D), jnp.float32)


def _tc1_body(x_ref, w1_ref, degp_ref, h1_ref):
    dinv = _dinv_cols(degp_ref[...])
    _scaled_matmul_pad(x_ref[...] * dinv, w1_ref[...], h1_ref)


def _block_tail(sp, hpad, x_in, degp, b, g, be):
    dinv = _dinv_cols(degp)
    s = sp[0, :N, :] + sp[1, :N, :]
    out = dinv * (s + hpad[:N, :]) + b[None, :]
    mean = jnp.mean(out, axis=0)
    var = jnp.mean((out - mean[None, :]) ** 2, axis=0)
    z = (out - mean[None, :]) * lax.rsqrt(var[None, :] + 1e-5) * g[None, :] + be[None, :]
    return _leaky(_leaky(z) + x_in), dinv


def _tc2_body(s1_ref, h1_ref, x_ref, degp_ref, b1_ref, g1_ref, be1_ref,
              w2_ref, x2_ref, h2_ref):
    x2, dinv = _block_tail(s1_ref[...], h1_ref[...], x_ref[...], degp_ref[...],
                           b1_ref[...], g1_ref[...], be1_ref[...])
    x2_ref[...] = x2
    _scaled_matmul_pad(x2 * dinv, w2_ref[...], h2_ref)


def _tc3_body(s2_ref, h2_ref, x2_ref, degp_ref, b2_ref, g2_ref, be2_ref,
              out_ref):
    h, _ = _block_tail(s2_ref[...], h2_ref[...], x2_ref[...], degp_ref[...],
                       b2_ref[...], g2_ref[...], be2_ref[...])
    nrm = jnp.maximum(jnp.sqrt(jnp.sum(h * h, axis=1, keepdims=True)), 1e-12)
    out_ref[...] = h / nrm


_tc1 = pl.pallas_call(
    _tc1_body,
    out_shape=jax.ShapeDtypeStruct((NPAD, D), jnp.float32),
)

_tc2 = pl.pallas_call(
    _tc2_body,
    out_shape=(jax.ShapeDtypeStruct((N, D), jnp.float32),
               jax.ShapeDtypeStruct((NPAD, D), jnp.float32)),
)

_tc3 = pl.pallas_call(
    _tc3_body,
    out_shape=jax.ShapeDtypeStruct((N, D), jnp.float32),
)


def kernel(x, edge_index, W1, b1, g1, be1, W2, b2, g2, be2):
    src = edge_index[0]
    dst = edge_index[1]
    pad = jnp.full((E_PAD - E,), N, jnp.int32)
    srcp = jnp.concatenate([src, pad])
    dstp = jnp.concatenate([dst, pad])
    ones_deg = jnp.ones((CHUNK, DEGC), jnp.float32)
    zeros_d = jnp.zeros((NPAD, D), jnp.float32)

    degp = _sc_degree(srcp, ones_deg, zeros_d)
    h1 = _tc1(x, W1, degp)
    s1 = _sc_messages(h1, srcp, dstp, zeros_d)
    x2, h2 = _tc2(s1, h1, x, degp, b1, g1, be1, W2)
    s2 = _sc_messages(h2, srcp, dstp, zeros_d)
    return _tc3(s2, h2, x2, degp, b2, g2, be2)
